# baseline (speedup 1.0000x reference)
import jax
import jax.numpy as jnp
from jax import lax
from jax.experimental import pallas as pl
from jax.experimental.pallas import tpu as pltpu

_ROWS = 16384
_BITS = 8
_GRID = 4


def _full_adder_bits(a, b, c):
    axb = a + b - 2.0 * a * b
    s = axb + c - 2.0 * axb * c
    ab = a * b
    cx = c * axb
    carry = ab + cx - ab * cx
    return s, carry


def _body(w1_ref, w2_ref, out_ref):
    subl = lax.broadcasted_iota(jnp.int32, (_BITS, 1), 0)
    c = jnp.float32(0.0)
    col = jnp.zeros((_BITS, 1), jnp.float32)
    for i in range(_BITS):
        a = (jnp.sign(w1_ref[0, i]) + 1.0) * 0.5
        b = (jnp.sign(w2_ref[0, i]) + 1.0) * 0.5
        s, c = _full_adder_bits(a, b, c)
        col = jnp.where(subl == i, s * 2.0 - 1.0, col)

    out_ref[...] = jnp.broadcast_to(col, (_BITS, _ROWS // _GRID))


def kernel(input, W1, W2):
    del input
    out_t = pl.pallas_call(
        _body,
        grid=(_GRID,),
        in_specs=[
            pl.BlockSpec(memory_space=pltpu.SMEM),
            pl.BlockSpec(memory_space=pltpu.SMEM),
        ],
        out_specs=pl.BlockSpec((_BITS, _ROWS // _GRID), lambda i: (0, i)),
        out_shape=jax.ShapeDtypeStruct((_BITS, _ROWS), jnp.float32),
    )(W1, W2)
    return out_t.T


# speedup vs baseline: 1.4393x; 1.4393x over previous
import jax
import jax.numpy as jnp
from jax import lax
from jax.experimental import pallas as pl
from jax.experimental.pallas import tpu as pltpu

_ROWS = 16384
_BITS = 8


def _full_adder_bits(a, b, c):
    axb = a + b - 2.0 * a * b
    s = axb + c - 2.0 * axb * c
    ab = a * b
    cx = c * axb
    carry = ab + cx - ab * cx
    return s, carry


def _body(w1_ref, w2_ref, out_ref):
    subl = lax.broadcasted_iota(jnp.int32, (_BITS, 1), 0)
    c = jnp.float32(0.0)
    col = jnp.zeros((_BITS, 1), jnp.float32)
    for i in range(_BITS):
        a = (jnp.sign(w1_ref[0, i]) + 1.0) * 0.5
        b = (jnp.sign(w2_ref[0, i]) + 1.0) * 0.5
        s, c = _full_adder_bits(a, b, c)
        col = jnp.where(subl == i, s * 2.0 - 1.0, col)

    out_ref[...] = jnp.broadcast_to(col, (_BITS, _ROWS))


def kernel(input, W1, W2):
    del input
    out_t = pl.pallas_call(
        _body,
        in_specs=[
            pl.BlockSpec(memory_space=pltpu.SMEM),
            pl.BlockSpec(memory_space=pltpu.SMEM),
        ],
        out_shape=jax.ShapeDtypeStruct((_BITS, _ROWS), jnp.float32),
    )(W1, W2)
    return out_t.T
